# baseline (speedup 1.0000x reference)
import functools

import jax
import jax.numpy as jnp
from jax import lax
from jax.experimental import pallas as pl
from jax.experimental.pallas import tpu as pltpu
from jax.experimental.pallas import tpu_sc as plsc

_NB_GOS = 100000
_D = 128
_B = 16384
_MARGIN = 0.1
_EPS = 1e-5

_NC = 2
_NS = 16
_NW = _NC * _NS
_BPW = _B // _NW


def _sc_gather_body(idx_c_hbm, idx_d_hbm, emb_hbm, rad_hbm,
                    c_out, d_out, rc_out, rd_out,
                    idx_v, rows_v, rad_v, sem):
    wid = lax.axis_index("s") * _NC + lax.axis_index("c")
    base = wid * _BPW
    pltpu.sync_copy(idx_c_hbm.at[pl.ds(base, _BPW)], idx_v)
    pltpu.async_copy(emb_hbm.at[idx_v], rows_v, sem).wait()
    pltpu.sync_copy(rows_v, c_out.at[pl.ds(base, _BPW)])
    pltpu.async_copy(rad_hbm.at[idx_v], rad_v, sem).wait()
    pltpu.sync_copy(rad_v, rc_out.at[pl.ds(base, _BPW)])
    pltpu.sync_copy(idx_d_hbm.at[pl.ds(base, _BPW)], idx_v)
    pltpu.async_copy(emb_hbm.at[idx_v], rows_v, sem).wait()
    pltpu.sync_copy(rows_v, d_out.at[pl.ds(base, _BPW)])
    pltpu.async_copy(rad_hbm.at[idx_v], rad_v, sem).wait()
    pltpu.sync_copy(rad_v, rd_out.at[pl.ds(base, _BPW)])


_sc_gather = pl.kernel(
    _sc_gather_body,
    out_type=(
        jax.ShapeDtypeStruct((_B, _D), jnp.float32),
        jax.ShapeDtypeStruct((_B, _D), jnp.float32),
        jax.ShapeDtypeStruct((_B, 1), jnp.float32),
        jax.ShapeDtypeStruct((_B, 1), jnp.float32),
    ),
    mesh=plsc.VectorSubcoreMesh(core_axis_name="c", subcore_axis_name="s"),
    compiler_params=pltpu.CompilerParams(use_tc_tiling_on_sc=False),
    scratch_types=(
        pltpu.VMEM((_BPW,), jnp.int32),
        pltpu.VMEM((_BPW, _D), jnp.float32),
        pltpu.VMEM((_BPW, 1), jnp.float32),
        pltpu.SemaphoreType.DMA,
    ),
)


def _tc_loss_body(c_ref, d_ref, rc_ref, rd_ref, g_ref, out_ref):
    c = c_ref[...]
    d = d_ref[...]
    n = jnp.float32(_B)
    mc = jnp.sum(c, axis=0, keepdims=True) / n
    vc = jnp.sum(c * c, axis=0, keepdims=True) / n - mc * mc
    md = jnp.sum(d, axis=0, keepdims=True) / n
    vd = jnp.sum(d * d, axis=0, keepdims=True) / n - md * md
    g = g_ref[...]
    a = g * lax.rsqrt(vc + _EPS)
    b = g * lax.rsqrt(vd + _EPS)
    diff = (c - mc) * a - (d - md) * b
    sq = jnp.sum(diff * diff, axis=1, keepdims=True)
    dist = jnp.sqrt(sq) + jnp.abs(rc_ref[...]) - jnp.abs(rd_ref[...])
    loss = jnp.sum(jnp.maximum(dist - _MARGIN, 0.0)) / n
    out_ref[0, 0] = loss


def kernel(data, go_embed_weight, go_rad_weight, bn_weight, bn_bias):
    del bn_bias
    idx_c = data[:, 0]
    idx_d = data[:, 1]
    c_raw, d_raw, rc, rd = _sc_gather(idx_c, idx_d, go_embed_weight,
                                      go_rad_weight)
    gamma = bn_weight.reshape(1, _D)
    loss = pl.pallas_call(
        _tc_loss_body,
        out_shape=jax.ShapeDtypeStruct((1, 1), jnp.float32),
        out_specs=pl.BlockSpec(memory_space=pltpu.SMEM),
    )(c_raw, d_raw, rc, rd, gamma)
    return loss.reshape(())
